# baseline (speedup 1.0000x reference)
import functools

import jax
import jax.numpy as jnp
from jax.experimental import pallas as pl
from jax.experimental.pallas import tpu as pltpu

NS, T, E, H = 512, 64, 300, 256
EP = 384
TSUB = 8
NT = T // TSUB


def _sigmoid(x):
    return jax.nn.sigmoid(x)


def _lstm_update(xt, h, c, Wih_ref, Whh_ref, b_ref):
    gates = (
        jnp.dot(xt.astype(jnp.bfloat16), Wih_ref[...],
                preferred_element_type=jnp.float32)
        + jnp.dot(h.astype(jnp.bfloat16), Whh_ref[...],
                  preferred_element_type=jnp.float32)
        + b_ref[...]
    )
    Hh = Whh_ref.shape[0]
    i = _sigmoid(gates[:, :Hh])
    f = _sigmoid(gates[:, Hh:2 * Hh])
    g = jnp.tanh(gates[:, 2 * Hh:3 * Hh])
    o = _sigmoid(gates[:, 3 * Hh:])
    c_new = f * c + i * g
    h_new = o * jnp.tanh(c_new)
    return h_new, c_new


def _pad_kernel(x_ref, o_ref):
    o_ref[:, :E] = x_ref[...].astype(jnp.bfloat16)
    o_ref[:, E:] = jnp.zeros((o_ref.shape[0], EP - E), jnp.bfloat16)


def _sent_kernel(lsm_ref, lens_ref, xf_ref, xb_ref, Wf_ref, Uf_ref, bf_ref,
                 Wb_ref, Ub_ref, bb_ref, out_ref, hf, cf, hb, cb):
    tt = pl.program_id(0)
    Lb = lsm_ref[0]

    @pl.when(tt == 0)
    def _init():
        hf[...] = jnp.zeros_like(hf)
        cf[...] = jnp.zeros_like(cf)
        hb[...] = jnp.zeros_like(hb)
        cb[...] = jnp.zeros_like(cb)

    lens = lens_ref[...]

    for k in range(TSUB):
        t = tt * TSUB + k

        @pl.when(t < Lb)
        def _fwd(k=k, t=t):
            xt = xf_ref[:, k * EP:(k + 1) * EP]
            h_new, c_new = _lstm_update(xt, hf[...], cf[...],
                                        Wf_ref, Uf_ref, bf_ref)
            m = t < lens
            hf[...] = jnp.where(m, h_new, hf[...])
            cf[...] = jnp.where(m, c_new, cf[...])

        tb = T - 1 - t

        @pl.when(tb < Lb)
        def _bwd(k=k, tb=tb):
            xt = xb_ref[:, (TSUB - 1 - k) * EP:(TSUB - k) * EP]
            h_new, c_new = _lstm_update(xt, hb[...], cb[...],
                                        Wb_ref, Ub_ref, bb_ref)
            mb = tb < lens
            hb[...] = jnp.where(mb, h_new, hb[...])
            cb[...] = jnp.where(mb, c_new, cb[...])

    @pl.when(tt == NT - 1)
    def _emit():
        out_ref[:, :H] = hf[...]
        out_ref[:, H:] = hb[...]


def _doc_kernel(ridx_ref, offs_ref, maxc_ref, cnts_ref, semb_ref,
                Wf_ref, Uf_ref, bf_ref, Wb_ref, Ub_ref, bb_ref,
                fcW_ref, fcb_ref, out_ref, xtf, xtb, hf, cf, hb, cb):
    B = cnts_ref.shape[0]
    maxc = maxc_ref[0]
    cnts = cnts_ref[...]

    hf[...] = jnp.zeros_like(hf)
    cf[...] = jnp.zeros_like(cf)
    hb[...] = jnp.zeros_like(hb)
    cb[...] = jnp.zeros_like(cb)

    def gather(t, dst):
        for d in range(B):
            addr = jnp.minimum(offs_ref[d] + t, NS - 1)
            j = ridx_ref[addr]
            dst[pl.ds(d, 1), :] = semb_ref[pl.ds(j, 1), :]

    def body(s, carry):
        gather(s, xtf)
        h_new, c_new = _lstm_update(xtf[...], hf[...], cf[...],
                                    Wf_ref, Uf_ref, bf_ref)
        m = s < cnts
        hf[...] = jnp.where(m, h_new, hf[...])
        cf[...] = jnp.where(m, c_new, cf[...])
        tb = maxc - 1 - s
        gather(tb, xtb)
        h_new, c_new = _lstm_update(xtb[...], hb[...], cb[...],
                                    Wb_ref, Ub_ref, bb_ref)
        mb = tb < cnts
        hb[...] = jnp.where(mb, h_new, hb[...])
        cb[...] = jnp.where(mb, c_new, cb[...])
        return carry

    jax.lax.fori_loop(0, maxc, body, 0, unroll=False)

    logits = (
        jnp.dot(hf[...], fcW_ref[:H, :], preferred_element_type=jnp.float32)
        + jnp.dot(hb[...], fcW_ref[H:, :], preferred_element_type=jnp.float32)
        + fcb_ref[0, 0]
    )
    out_ref[...] = _sigmoid(logits)


@jax.jit
def kernel(x, sWihf, sWhhf, sbf, sWihb, sWhhb, sbb, dWihf, dWhhf, dbf,
           dWihb, dWhhb, dbb, fcW, fcb, recover_idx, num_sent_per_document,
           sent_lengths):
    B = num_sent_per_document.shape[0]
    lens2d = sent_lengths.reshape(NS, 1)

    xp = pl.pallas_call(
        _pad_kernel,
        grid=(T,),
        in_specs=[pl.BlockSpec((NS, E), lambda i: (i, 0))],
        out_specs=pl.BlockSpec((NS, EP), lambda i: (i, 0)),
        out_shape=jax.ShapeDtypeStruct((NS * T, EP), jnp.bfloat16),
    )(x.reshape(NS * T, E))
    xp = xp.reshape(NS, T * EP)

    def padW(W):
        return jnp.pad(W, ((0, EP - E), (0, 0))).astype(jnp.bfloat16)

    sent_emb_sorted = pl.pallas_call(
        _sent_kernel,
        grid=(NT,),
        in_specs=[
            pl.BlockSpec(memory_space=pltpu.SMEM),
            pl.BlockSpec((NS, 1), lambda tt: (0, 0)),
            pl.BlockSpec((NS, TSUB * EP), lambda tt: (0, tt)),
            pl.BlockSpec((NS, TSUB * EP), lambda tt: (0, NT - 1 - tt)),
            pl.BlockSpec((EP, 4 * H), lambda tt: (0, 0)),
            pl.BlockSpec((H, 4 * H), lambda tt: (0, 0)),
            pl.BlockSpec((1, 4 * H), lambda tt: (0, 0)),
            pl.BlockSpec((EP, 4 * H), lambda tt: (0, 0)),
            pl.BlockSpec((H, 4 * H), lambda tt: (0, 0)),
            pl.BlockSpec((1, 4 * H), lambda tt: (0, 0)),
        ],
        out_specs=pl.BlockSpec((NS, 2 * H), lambda tt: (0, 0)),
        out_shape=jax.ShapeDtypeStruct((NS, 2 * H), jnp.float32),
        scratch_shapes=[pltpu.VMEM((NS, H), jnp.float32)] * 4,
    )(sent_lengths.astype(jnp.int32), lens2d, xp, xp,
      padW(sWihf), sWhhf.astype(jnp.bfloat16), sbf.reshape(1, -1),
      padW(sWihb), sWhhb.astype(jnp.bfloat16), sbb.reshape(1, -1))

    counts = num_sent_per_document.astype(jnp.int32)
    offsets = jnp.concatenate(
        [jnp.zeros((1,), jnp.int32), jnp.cumsum(counts)[:-1]])
    maxc = jnp.max(counts).reshape(1)

    out2d = pl.pallas_call(
        _doc_kernel,
        in_specs=[
            pl.BlockSpec(memory_space=pltpu.SMEM),
            pl.BlockSpec(memory_space=pltpu.SMEM),
            pl.BlockSpec(memory_space=pltpu.SMEM),
            pl.BlockSpec((B, 1), lambda: (0, 0)),
            pl.BlockSpec((NS, 2 * H), lambda: (0, 0)),
            pl.BlockSpec((2 * H, 4 * H), lambda: (0, 0)),
            pl.BlockSpec((H, 4 * H), lambda: (0, 0)),
            pl.BlockSpec((1, 4 * H), lambda: (0, 0)),
            pl.BlockSpec((2 * H, 4 * H), lambda: (0, 0)),
            pl.BlockSpec((H, 4 * H), lambda: (0, 0)),
            pl.BlockSpec((1, 4 * H), lambda: (0, 0)),
            pl.BlockSpec((2 * H, 1), lambda: (0, 0)),
            pl.BlockSpec((1, 1), lambda: (0, 0)),
        ],
        out_specs=pl.BlockSpec((B, 1), lambda: (0, 0)),
        out_shape=jax.ShapeDtypeStruct((B, 1), jnp.float32),
        scratch_shapes=[pltpu.VMEM((B, 2 * H), jnp.float32)] * 2
        + [pltpu.VMEM((B, H), jnp.float32)] * 4,
    )(recover_idx.astype(jnp.int32), offsets, maxc, counts.reshape(B, 1),
      sent_emb_sorted, dWihf.astype(jnp.bfloat16), dWhhf.astype(jnp.bfloat16),
      dbf.reshape(1, -1), dWihb.astype(jnp.bfloat16),
      dWhhb.astype(jnp.bfloat16), dbb.reshape(1, -1), fcW, fcb.reshape(1, 1))

    return out2d.reshape(-1)


# speedup vs baseline: 1.1563x; 1.1563x over previous
import functools

import jax
import jax.numpy as jnp
from jax.experimental import pallas as pl
from jax.experimental.pallas import tpu as pltpu

NS, T, E, H = 512, 64, 300, 256
EP = 384
TSUB = 8
NT = T // TSUB


def _sigmoid(x):
    return 0.5 * jnp.tanh(0.5 * x) + 0.5


def _gates_update(gates, c, Hh):
    i = _sigmoid(gates[:, :Hh])
    f = _sigmoid(gates[:, Hh:2 * Hh])
    g = jnp.tanh(gates[:, 2 * Hh:3 * Hh])
    o = _sigmoid(gates[:, 3 * Hh:])
    c_new = f * c + i * g
    h_new = o * jnp.tanh(c_new)
    return h_new, c_new


def _lstm_update(xt, h, c, Wih_ref, Whh_ref, b_ref):
    gates = (
        jnp.dot(xt.astype(jnp.bfloat16), Wih_ref[...],
                preferred_element_type=jnp.float32)
        + jnp.dot(h.astype(jnp.bfloat16), Whh_ref[...],
                  preferred_element_type=jnp.float32)
        + b_ref[...]
    )
    return _gates_update(gates, c, Whh_ref.shape[0])


def _lstm_update_cat(xt, h, c, W_ref, b_ref, Hh):
    xh = jnp.concatenate([xt, h.astype(jnp.bfloat16)], axis=1)
    gates = jnp.dot(xh, W_ref[...], preferred_element_type=jnp.float32) + b_ref[...]
    return _gates_update(gates, c, Hh)


def _pad_kernel(x_ref, o_ref):
    for k in range(TSUB):
        o_ref[:, k * EP:k * EP + E] = x_ref[:, k, :].astype(jnp.bfloat16)
        o_ref[:, k * EP + E:(k + 1) * EP] = jnp.zeros(
            (o_ref.shape[0], EP - E), jnp.bfloat16)


def _sent_kernel(lsm_ref, lens_ref, xf_ref, xb_ref, Wf_ref, bf_ref,
                 Wb_ref, bb_ref, out_ref, hf, cf, hb, cb):
    tt = pl.program_id(0)
    Lb = lsm_ref[0]

    @pl.when(tt == 0)
    def _init():
        hf[...] = jnp.zeros_like(hf)
        cf[...] = jnp.zeros_like(cf)
        hb[...] = jnp.zeros_like(hb)
        cb[...] = jnp.zeros_like(cb)

    lens = lens_ref[...]

    for k in range(TSUB):
        t = tt * TSUB + k

        @pl.when(t < Lb)
        def _fwd(k=k, t=t):
            xt = xf_ref[:, k * EP:(k + 1) * EP]
            h_new, c_new = _lstm_update_cat(xt, hf[...], cf[...],
                                            Wf_ref, bf_ref, H)
            m = t < lens
            hf[...] = jnp.where(m, h_new, hf[...])
            cf[...] = jnp.where(m, c_new, cf[...])

        tb = T - 1 - t

        @pl.when(tb < Lb)
        def _bwd(k=k, tb=tb):
            xt = xb_ref[:, (TSUB - 1 - k) * EP:(TSUB - k) * EP]
            h_new, c_new = _lstm_update_cat(xt, hb[...], cb[...],
                                            Wb_ref, bb_ref, H)
            mb = tb < lens
            hb[...] = jnp.where(mb, h_new, hb[...])
            cb[...] = jnp.where(mb, c_new, cb[...])

    @pl.when(tt == NT - 1)
    def _emit():
        out_ref[:, :H] = hf[...]
        out_ref[:, H:] = hb[...]


def _doc_kernel(ridx_ref, offs_ref, maxc_ref, cnts_ref, semb_ref,
                Wf_ref, Uf_ref, bf_ref, Wb_ref, Ub_ref, bb_ref,
                fcW_ref, fcb_ref, out_ref, xtf, xtb, hf, cf, hb, cb):
    B = cnts_ref.shape[0]
    maxc = maxc_ref[0]
    cnts = cnts_ref[...]

    hf[...] = jnp.zeros_like(hf)
    cf[...] = jnp.zeros_like(cf)
    hb[...] = jnp.zeros_like(hb)
    cb[...] = jnp.zeros_like(cb)

    def gather(t, dst):
        for d in range(B):
            addr = jnp.minimum(offs_ref[d] + t, NS - 1)
            j = ridx_ref[addr]
            dst[pl.ds(d, 1), :] = semb_ref[pl.ds(j, 1), :]

    def body(s, carry):
        gather(s, xtf)
        h_new, c_new = _lstm_update(xtf[...], hf[...], cf[...],
                                    Wf_ref, Uf_ref, bf_ref)
        m = s < cnts
        hf[...] = jnp.where(m, h_new, hf[...])
        cf[...] = jnp.where(m, c_new, cf[...])
        tb = maxc - 1 - s
        gather(tb, xtb)
        h_new, c_new = _lstm_update(xtb[...], hb[...], cb[...],
                                    Wb_ref, Ub_ref, bb_ref)
        mb = tb < cnts
        hb[...] = jnp.where(mb, h_new, hb[...])
        cb[...] = jnp.where(mb, c_new, cb[...])
        return carry

    jax.lax.fori_loop(0, maxc, body, 0, unroll=False)

    logits = (
        jnp.dot(hf[...], fcW_ref[:H, :], preferred_element_type=jnp.float32)
        + jnp.dot(hb[...], fcW_ref[H:, :], preferred_element_type=jnp.float32)
        + fcb_ref[0, 0]
    )
    out_ref[...] = _sigmoid(logits)


@jax.jit
def kernel(x, sWihf, sWhhf, sbf, sWihb, sWhhb, sbb, dWihf, dWhhf, dbf,
           dWihb, dWhhb, dbb, fcW, fcb, recover_idx, num_sent_per_document,
           sent_lengths):
    B = num_sent_per_document.shape[0]
    lens2d = sent_lengths.reshape(NS, 1)

    xp = pl.pallas_call(
        _pad_kernel,
        grid=(NT,),
        in_specs=[pl.BlockSpec((NS, TSUB, E), lambda i: (0, i, 0))],
        out_specs=pl.BlockSpec((NS, TSUB * EP), lambda i: (0, i)),
        out_shape=jax.ShapeDtypeStruct((NS, T * EP), jnp.bfloat16),
    )(x)

    def catW(Wih, Whh):
        return jnp.concatenate(
            [jnp.pad(Wih, ((0, EP - E), (0, 0))), Whh]).astype(jnp.bfloat16)

    sent_emb_sorted = pl.pallas_call(
        _sent_kernel,
        grid=(NT,),
        in_specs=[
            pl.BlockSpec(memory_space=pltpu.SMEM),
            pl.BlockSpec((NS, 1), lambda tt: (0, 0)),
            pl.BlockSpec((NS, TSUB * EP), lambda tt: (0, tt)),
            pl.BlockSpec((NS, TSUB * EP), lambda tt: (0, NT - 1 - tt)),
            pl.BlockSpec((EP + H, 4 * H), lambda tt: (0, 0)),
            pl.BlockSpec((1, 4 * H), lambda tt: (0, 0)),
            pl.BlockSpec((EP + H, 4 * H), lambda tt: (0, 0)),
            pl.BlockSpec((1, 4 * H), lambda tt: (0, 0)),
        ],
        out_specs=pl.BlockSpec((NS, 2 * H), lambda tt: (0, 0)),
        out_shape=jax.ShapeDtypeStruct((NS, 2 * H), jnp.float32),
        scratch_shapes=[pltpu.VMEM((NS, H), jnp.float32)] * 4,
    )(sent_lengths.astype(jnp.int32), lens2d, xp, xp,
      catW(sWihf, sWhhf), sbf.reshape(1, -1),
      catW(sWihb, sWhhb), sbb.reshape(1, -1))

    counts = num_sent_per_document.astype(jnp.int32)
    offsets = jnp.concatenate(
        [jnp.zeros((1,), jnp.int32), jnp.cumsum(counts)[:-1]])
    maxc = jnp.max(counts).reshape(1)

    out2d = pl.pallas_call(
        _doc_kernel,
        in_specs=[
            pl.BlockSpec(memory_space=pltpu.SMEM),
            pl.BlockSpec(memory_space=pltpu.SMEM),
            pl.BlockSpec(memory_space=pltpu.SMEM),
            pl.BlockSpec((B, 1), lambda: (0, 0)),
            pl.BlockSpec((NS, 2 * H), lambda: (0, 0)),
            pl.BlockSpec((2 * H, 4 * H), lambda: (0, 0)),
            pl.BlockSpec((H, 4 * H), lambda: (0, 0)),
            pl.BlockSpec((1, 4 * H), lambda: (0, 0)),
            pl.BlockSpec((2 * H, 4 * H), lambda: (0, 0)),
            pl.BlockSpec((H, 4 * H), lambda: (0, 0)),
            pl.BlockSpec((1, 4 * H), lambda: (0, 0)),
            pl.BlockSpec((2 * H, 1), lambda: (0, 0)),
            pl.BlockSpec((1, 1), lambda: (0, 0)),
        ],
        out_specs=pl.BlockSpec((B, 1), lambda: (0, 0)),
        out_shape=jax.ShapeDtypeStruct((B, 1), jnp.float32),
        scratch_shapes=[pltpu.VMEM((B, 2 * H), jnp.float32)] * 2
        + [pltpu.VMEM((B, H), jnp.float32)] * 4,
    )(recover_idx.astype(jnp.int32), offsets, maxc, counts.reshape(B, 1),
      sent_emb_sorted, dWihf.astype(jnp.bfloat16), dWhhf.astype(jnp.bfloat16),
      dbf.reshape(1, -1), dWihb.astype(jnp.bfloat16),
      dWhhb.astype(jnp.bfloat16), dbb.reshape(1, -1), fcW, fcb.reshape(1, 1))

    return out2d.reshape(-1)


# speedup vs baseline: 1.3846x; 1.1974x over previous
import functools

import jax
import jax.numpy as jnp
from jax.experimental import pallas as pl
from jax.experimental.pallas import tpu as pltpu

NS, T, E, H = 512, 64, 300, 256
TSUB = 8
NT = T // TSUB
LOOKAHEAD = 3
DEPTH = 4


def _sigmoid(x):
    return 0.5 * jnp.tanh(0.5 * x) + 0.5


def _gates_update(gates, c, Hh):
    i = _sigmoid(gates[:, :Hh])
    f = _sigmoid(gates[:, Hh:2 * Hh])
    g = jnp.tanh(gates[:, 2 * Hh:3 * Hh])
    o = _sigmoid(gates[:, 3 * Hh:])
    c_new = f * c + i * g
    h_new = o * jnp.tanh(c_new)
    return h_new, c_new


def _lstm_update(xt, h, c, Wih_ref, Whh_ref, b_ref):
    gates = (
        jnp.dot(xt.astype(jnp.bfloat16), Wih_ref[...],
                preferred_element_type=jnp.float32)
        + jnp.dot(h.astype(jnp.bfloat16), Whh_ref[...],
                  preferred_element_type=jnp.float32)
        + b_ref[...]
    )
    return _gates_update(gates, c, Whh_ref.shape[0])


def _lstm_update_cat(xt, h, c, W_ref, b_ref, Hh):
    xh = jnp.concatenate([xt.astype(jnp.bfloat16), h.astype(jnp.bfloat16)],
                         axis=1)
    gates = jnp.dot(xh, W_ref[...], preferred_element_type=jnp.float32) + b_ref[...]
    return _gates_update(gates, c, Hh)


def _xcopy(x_ref, buf, sem, t, slot):
    return pltpu.make_async_copy(x_ref.at[:, t, :], buf.at[slot], sem.at[slot])


def _sent_kernel(lens_ref, x_ref, Wf_ref, Uf_ref, bf_ref, Wb_ref, Ub_ref,
                 bb_ref, out_ref, hf, cf, hb, cb, fbuf, bbuf, fsem, bsem):
    tt = pl.program_id(0)

    @pl.when(tt == 0)
    def _init():
        hf[...] = jnp.zeros_like(hf)
        cf[...] = jnp.zeros_like(cf)
        hb[...] = jnp.zeros_like(hb)
        cb[...] = jnp.zeros_like(cb)
        for u in range(LOOKAHEAD):
            _xcopy(x_ref, fbuf, fsem, u, u % DEPTH).start()
            _xcopy(x_ref, bbuf, bsem, T - 1 - u, u % DEPTH).start()

    lens = lens_ref[...]

    for k in range(TSUB):
        t = tt * TSUB + k
        tb = T - 1 - t

        tpre = t + LOOKAHEAD

        @pl.when(tpre < T)
        def _pre(tpre=tpre):
            _xcopy(x_ref, fbuf, fsem, tpre, tpre % DEPTH).start()
            _xcopy(x_ref, bbuf, bsem, T - 1 - tpre, tpre % DEPTH).start()

        _xcopy(x_ref, fbuf, fsem, t, t % DEPTH).wait()
        xt = fbuf[t % DEPTH]
        h_new, c_new = _lstm_update(xt, hf[...], cf[...],
                                    Wf_ref, Uf_ref, bf_ref)
        m = t < lens
        hf[...] = jnp.where(m, h_new, hf[...])
        cf[...] = jnp.where(m, c_new, cf[...])

        _xcopy(x_ref, bbuf, bsem, tb, t % DEPTH).wait()
        xtb = bbuf[t % DEPTH]
        h_new, c_new = _lstm_update(xtb, hb[...], cb[...],
                                    Wb_ref, Ub_ref, bb_ref)
        mb = tb < lens
        hb[...] = jnp.where(mb, h_new, hb[...])
        cb[...] = jnp.where(mb, c_new, cb[...])

    @pl.when(tt == NT - 1)
    def _emit():
        out_ref[:, :H] = hf[...]
        out_ref[:, H:] = hb[...]


def _doc_kernel(ridx_ref, offs_ref, maxc_ref, cnts_ref, semb_ref,
                Wf_ref, bf_ref, Wb_ref, bb_ref,
                fcW_ref, fcb_ref, out_ref, P, hf, cf, hb, cb):
    B = cnts_ref.shape[0]
    maxc = maxc_ref[0]
    cnts = cnts_ref[...]

    hf[...] = jnp.zeros_like(hf)
    cf[...] = jnp.zeros_like(cf)
    hb[...] = jnp.zeros_like(hb)
    cb[...] = jnp.zeros_like(cb)

    def stage(t, carry):
        for d in range(B):
            addr = jnp.minimum(offs_ref[d] + t, NS - 1)
            j = ridx_ref[addr]
            P[t, pl.ds(d, 1), :] = semb_ref[pl.ds(j, 1), :].astype(
                jnp.bfloat16)
        return carry

    jax.lax.fori_loop(0, maxc, stage, 0, unroll=False)

    def body(s, carry):
        xt = P[s]
        h_new, c_new = _lstm_update_cat(xt, hf[...], cf[...],
                                        Wf_ref, bf_ref, H)
        m = s < cnts
        hf[...] = jnp.where(m, h_new, hf[...])
        cf[...] = jnp.where(m, c_new, cf[...])
        tb = maxc - 1 - s
        xtb = P[tb]
        h_new, c_new = _lstm_update_cat(xtb, hb[...], cb[...],
                                        Wb_ref, bb_ref, H)
        mb = tb < cnts
        hb[...] = jnp.where(mb, h_new, hb[...])
        cb[...] = jnp.where(mb, c_new, cb[...])
        return carry

    jax.lax.fori_loop(0, maxc, body, 0, unroll=False)

    logits = (
        jnp.dot(hf[...], fcW_ref[:H, :], preferred_element_type=jnp.float32)
        + jnp.dot(hb[...], fcW_ref[H:, :], preferred_element_type=jnp.float32)
        + fcb_ref[0, 0]
    )
    out_ref[...] = _sigmoid(logits)


@jax.jit
def kernel(x, sWihf, sWhhf, sbf, sWihb, sWhhb, sbb, dWihf, dWhhf, dbf,
           dWihb, dWhhb, dbb, fcW, fcb, recover_idx, num_sent_per_document,
           sent_lengths):
    B = num_sent_per_document.shape[0]
    lens2d = sent_lengths.reshape(NS, 1)

    sent_emb_sorted = pl.pallas_call(
        _sent_kernel,
        grid=(NT,),
        in_specs=[
            pl.BlockSpec((NS, 1), lambda tt: (0, 0)),
            pl.BlockSpec(memory_space=pltpu.MemorySpace.HBM),
            pl.BlockSpec((E, 4 * H), lambda tt: (0, 0)),
            pl.BlockSpec((H, 4 * H), lambda tt: (0, 0)),
            pl.BlockSpec((1, 4 * H), lambda tt: (0, 0)),
            pl.BlockSpec((E, 4 * H), lambda tt: (0, 0)),
            pl.BlockSpec((H, 4 * H), lambda tt: (0, 0)),
            pl.BlockSpec((1, 4 * H), lambda tt: (0, 0)),
        ],
        out_specs=pl.BlockSpec((NS, 2 * H), lambda tt: (0, 0)),
        out_shape=jax.ShapeDtypeStruct((NS, 2 * H), jnp.float32),
        scratch_shapes=[pltpu.VMEM((NS, H), jnp.float32)] * 4
        + [pltpu.VMEM((DEPTH, NS, E), jnp.float32)] * 2
        + [pltpu.SemaphoreType.DMA((DEPTH,))] * 2,
    )(lens2d, x,
      sWihf.astype(jnp.bfloat16), sWhhf.astype(jnp.bfloat16),
      sbf.reshape(1, -1),
      sWihb.astype(jnp.bfloat16), sWhhb.astype(jnp.bfloat16),
      sbb.reshape(1, -1))

    counts = num_sent_per_document.astype(jnp.int32)
    offsets = jnp.concatenate(
        [jnp.zeros((1,), jnp.int32), jnp.cumsum(counts)[:-1]])
    maxc = jnp.max(counts).reshape(1)

    def catW(Wih, Whh):
        return jnp.concatenate([Wih, Whh]).astype(jnp.bfloat16)

    out2d = pl.pallas_call(
        _doc_kernel,
        in_specs=[
            pl.BlockSpec(memory_space=pltpu.SMEM),
            pl.BlockSpec(memory_space=pltpu.SMEM),
            pl.BlockSpec(memory_space=pltpu.SMEM),
            pl.BlockSpec((B, 1), lambda: (0, 0)),
            pl.BlockSpec((NS, 2 * H), lambda: (0, 0)),
            pl.BlockSpec((3 * H, 4 * H), lambda: (0, 0)),
            pl.BlockSpec((1, 4 * H), lambda: (0, 0)),
            pl.BlockSpec((3 * H, 4 * H), lambda: (0, 0)),
            pl.BlockSpec((1, 4 * H), lambda: (0, 0)),
            pl.BlockSpec((2 * H, 1), lambda: (0, 0)),
            pl.BlockSpec((1, 1), lambda: (0, 0)),
        ],
        out_specs=pl.BlockSpec((B, 1), lambda: (0, 0)),
        out_shape=jax.ShapeDtypeStruct((B, 1), jnp.float32),
        scratch_shapes=[pltpu.VMEM((NS, B, 2 * H), jnp.bfloat16)]
        + [pltpu.VMEM((B, H), jnp.float32)] * 4,
    )(recover_idx.astype(jnp.int32), offsets, maxc, counts.reshape(B, 1),
      sent_emb_sorted,
      catW(dWihf, dWhhf), dbf.reshape(1, -1),
      catW(dWihb, dWhhb), dbb.reshape(1, -1), fcW, fcb.reshape(1, 1))

    return out2d.reshape(-1)
